# baseline (speedup 1.0000x reference)
import functools

import jax
import jax.numpy as jnp
from jax import lax
from jax.experimental import pallas as pl
from jax.experimental.pallas import tpu as pltpu
from jax.experimental.pallas import tpu_sc as plsc

BATCH = 16384
F = 26
VOCAB = 1000000
VOCAB_PAD = 1000448
NC = 2
NS = 16
L = 16
NW = NC * NS
BW = BATCH // NW
NG = F * BW


def _lr_body(idx_hbm, table_hbm, out_hbm, idx_v, vals_v, out_v, sem):
    wid = lax.axis_index("s") * NC + lax.axis_index("c")
    base = wid * NG
    with jax.named_scope("idx_stage"):
        pltpu.sync_copy(idx_hbm.at[pl.ds(base, NG)], idx_v)
    with jax.named_scope("gather"):
        pltpu.async_copy(table_hbm.at[idx_v], vals_v, sem).wait()

    def chunk(c, carry):
        off = c * L
        acc = vals_v[pl.ds(off, L)]
        for f in range(1, F):
            acc = acc + vals_v[pl.ds(f * BW + off, L)]
        x = acc * (1.0 / F)
        out_v[pl.ds(off, L)] = 1.0 / (1.0 + jnp.exp(-x))
        return carry

    with jax.named_scope("reduce_sigmoid"):
        lax.fori_loop(0, BW // L, chunk, 0)
    with jax.named_scope("out_store"):
        pltpu.sync_copy(out_v, out_hbm.at[pl.ds(wid * BW, BW)])


_lr_call = functools.partial(
    pl.kernel,
    mesh=plsc.VectorSubcoreMesh(core_axis_name="c", subcore_axis_name="s"),
    out_type=jax.ShapeDtypeStruct((BATCH,), jnp.float32),
    scratch_types=[
        pltpu.VMEM((NG,), jnp.int32),
        pltpu.VMEM((NG,), jnp.float32),
        pltpu.VMEM((BW,), jnp.float32),
        pltpu.SemaphoreType.DMA,
    ],
    compiler_params=pltpu.CompilerParams(needs_layout_passes=False),
)(_lr_body)


@jax.jit
def kernel(inputs, table):
    idx = inputs.astype(jnp.int32).reshape(NW, BW, F).transpose(0, 2, 1).reshape(-1)
    tflat = jnp.pad(table, ((0, VOCAB_PAD - VOCAB), (0, 0))).reshape(-1)
    out = _lr_call(idx, tflat)
    return out.reshape(BATCH, 1)


# speedup vs baseline: 1.0050x; 1.0050x over previous
import functools

import jax
import jax.numpy as jnp
from jax import lax
from jax.experimental import pallas as pl
from jax.experimental.pallas import tpu as pltpu
from jax.experimental.pallas import tpu_sc as plsc

BATCH = 16384
F = 26
VOCAB = 1000000
VOCAB_PAD = 1000448
NC = 2
NS = 16
L = 16
NW = NC * NS
BW = BATCH // NW
NG = F * BW


def _lr_body(idx_hbm, table_hbm, out_hbm, idx_v, vals_v, out_v, sem):
    wid = lax.axis_index("s") * NC + lax.axis_index("c")
    base = wid * NG
    pltpu.sync_copy(idx_hbm.at[pl.ds(base, NG)], idx_v)
    pltpu.async_copy(table_hbm.at[idx_v], vals_v, sem).wait()

    def chunk(c, carry):
        off = c * L
        acc = vals_v[pl.ds(off, L)]
        for f in range(1, F):
            acc = acc + vals_v[pl.ds(f * BW + off, L)]
        x = acc * (1.0 / F)
        out_v[pl.ds(off, L)] = 1.0 / (1.0 + jnp.exp(-x))
        return carry

    lax.fori_loop(0, BW // L, chunk, 0)
    pltpu.sync_copy(out_v, out_hbm.at[pl.ds(wid * BW, BW)])


_lr_call = functools.partial(
    pl.kernel,
    mesh=plsc.VectorSubcoreMesh(core_axis_name="c", subcore_axis_name="s"),
    out_type=jax.ShapeDtypeStruct((BATCH,), jnp.float32),
    scratch_types=[
        pltpu.VMEM((NG,), jnp.int32),
        pltpu.VMEM((NG,), jnp.float32),
        pltpu.VMEM((BW,), jnp.float32),
        pltpu.SemaphoreType.DMA,
    ],
    compiler_params=pltpu.CompilerParams(needs_layout_passes=False),
)(_lr_body)


@jax.jit
def kernel(inputs, table):
    idx = inputs.astype(jnp.int32).reshape(NW, BW, F).transpose(0, 2, 1).reshape(-1)
    tflat = jnp.pad(table, ((0, VOCAB_PAD - VOCAB), (0, 0))).reshape(-1)
    out = _lr_call(idx, tflat)
    return out.reshape(BATCH, 1)
